# baseline (speedup 1.0000x reference)
import functools

import numpy as np
import jax
import jax.numpy as jnp
from jax import lax
from jax.experimental import pallas as pl
from jax.experimental.pallas import tpu as pltpu
from jax.experimental.pallas import tpu_sc as plsc

B = 4
C = 768
HW = 576
D = 256
K = 32
NEG = -1e9
SCALE_INV = 1.0 / 16.0
EPS = 1e-5
ROWS = B * C

_HIGH = jax.lax.Precision.HIGHEST


def _resize_matrix(out_n: int, in_n: int) -> np.ndarray:
    R = np.zeros((out_n, in_n), np.float32)
    for i in range(out_n):
        s = (i + 0.5) * in_n / out_n - 0.5
        i0 = int(np.floor(s))
        f = s - i0
        R[i, min(max(i0, 0), in_n - 1)] += 1.0 - f
        R[i, min(max(i0 + 1, 0), in_n - 1)] += f
    return R


_R24 = _resize_matrix(24, 12)
_KRT = np.kron(_R24, _R24).T.astype(np.float32)


def _tc1_body(x1_ref, x2_ref, krt_ref, wq_ref, wk_ref, bq_ref, bk_ref,
              xf_ref, att_ref, diag_ref):
    xa = x1_ref[0]
    xb = jnp.dot(x2_ref[0], krt_ref[...],
                 preferred_element_type=jnp.float32, precision=_HIGH)
    xf = jnp.concatenate([xa, xb], axis=0)
    xf_ref[0] = xf
    q = jnp.dot(xf, wq_ref[...],
                preferred_element_type=jnp.float32) + bq_ref[...]
    k = jnp.dot(xf, wk_ref[...],
                preferred_element_type=jnp.float32) + bk_ref[...]
    att = lax.dot_general(q, k, (((1,), (1,)), ((), ())),
                          preferred_element_type=jnp.float32) * SCALE_INV
    ri = lax.broadcasted_iota(jnp.int32, (C, C), 0)
    ci = lax.broadcasted_iota(jnp.int32, (C, C), 1)
    eye = ri == ci
    diag_ref[0] = jnp.sum(jnp.where(eye, att, 0.0), axis=0, keepdims=True)
    att_ref[0] = jnp.where(eye, NEG, att)


def _tc1():
    return pl.pallas_call(
        _tc1_body,
        grid=(B,),
        in_specs=[
            pl.BlockSpec((1, 384, HW), lambda b: (b, 0, 0)),
            pl.BlockSpec((1, 384, 144), lambda b: (b, 0, 0)),
            pl.BlockSpec((144, HW), lambda b: (0, 0)),
            pl.BlockSpec((HW, D), lambda b: (0, 0)),
            pl.BlockSpec((HW, D), lambda b: (0, 0)),
            pl.BlockSpec((1, D), lambda b: (0, 0)),
            pl.BlockSpec((1, D), lambda b: (0, 0)),
        ],
        out_specs=[
            pl.BlockSpec((1, C, HW), lambda b: (b, 0, 0)),
            pl.BlockSpec((1, C, C), lambda b: (b, 0, 0)),
            pl.BlockSpec((1, 1, C), lambda b: (b, 0, 0)),
        ],
        out_shape=[
            jax.ShapeDtypeStruct((B, C, HW), jnp.float32),
            jax.ShapeDtypeStruct((B, C, C), jnp.float32),
            jax.ShapeDtypeStruct((B, 1, C), jnp.float32),
        ],
    )


_NC = 2
_NS = 16
_NW = _NC * _NS
_RPW = ROWS // _NW


def _sort16a(v):
    return lax.sort(v, dimension=0)


def _sort16d(v):
    return -lax.sort(-v, dimension=0)


def _leafpair(a, b, order):
    A = _sort16a(a)
    Bd = _sort16d(b)
    lo = jnp.minimum(A, Bd)
    hi = jnp.maximum(A, Bd)
    if order > 0:
        return _sort16a(lo), _sort16a(hi)
    return _sort16d(hi), _sort16d(lo)


def _merge32(A, Bd, order):
    H0 = jnp.maximum(A[0], Bd[0])
    H1 = jnp.maximum(A[1], Bd[1])
    P = jnp.minimum(H0, H1)
    Q = jnp.maximum(H0, H1)
    if order > 0:
        return _sort16a(P), _sort16a(Q)
    return _sort16d(Q), _sort16d(P)


def _reduce_nodes(builders, order):
    if len(builders) == 1:
        return builders[0](order)
    mid = (len(builders) + 1) // 2
    left = _reduce_nodes(builders[:mid], +1)
    right = _reduce_nodes(builders[mid:], -1)
    return _merge32(left, right, order)


def _row_threshold(att_v, r):
    chunks = [att_v[r, pl.ds(16 * j, 16)] for j in range(C // 16)]
    builders = [
        (lambda o, i=i: _leafpair(chunks[2 * i], chunks[2 * i + 1], o))
        for i in range(len(chunks) // 2)
    ]
    A = _reduce_nodes(builders[:12], +1)
    Bd = _reduce_nodes(builders[12:], -1)
    H0 = jnp.maximum(A[0], Bd[0])
    H1 = jnp.maximum(A[1], Bd[1])
    return jnp.min(jnp.minimum(H0, H1))


def _sc_topk_kernel(att_hbm, t_hbm, att_v, t_v):
    wid = lax.axis_index("s") * _NC + lax.axis_index("c")
    base = wid * _RPW
    pltpu.sync_copy(att_hbm.at[pl.ds(base, _RPW), :], att_v)

    def row_body(i, carry):
        for u in range(2):
            r = i * 2 + u
            t = _row_threshold(att_v, r)
            t_v[pl.ds(r * 128, 16)] = jnp.full((16,), t, jnp.float32)
        return carry

    lax.fori_loop(0, _RPW // 2, row_body, 0)
    pltpu.sync_copy(t_v, t_hbm.at[pl.ds(base * 128, _RPW * 128)])


def _sc_topk(att_flat):
    fn = functools.partial(
        pl.kernel,
        mesh=plsc.VectorSubcoreMesh(core_axis_name="c", subcore_axis_name="s"),
        compiler_params=pltpu.CompilerParams(needs_layout_passes=False,
                                             use_tc_tiling_on_sc=True),
        out_type=jax.ShapeDtypeStruct((ROWS * 128,), jnp.float32),
        scratch_types=[
            pltpu.VMEM((_RPW, C), jnp.float32),
            pltpu.VMEM((_RPW * 128,), jnp.float32),
        ],
    )(_sc_topk_kernel)
    return fn(att_flat)


def _tc2_body(att_ref, t_ref, diag_ref, xf_ref, cw_ref, g_ref, be_ref,
              o_ref, y_scr):
    ri = lax.broadcasted_iota(jnp.int32, (C, C), 0)
    ci = lax.broadcasted_iota(jnp.int32, (C, C), 1)
    eye = ri == ci
    tot = jnp.zeros((D, 1), jnp.float32)
    tot2 = jnp.zeros((D, 1), jnp.float32)
    for b in range(B):
        att = att_ref[b]
        tcol = t_ref[b][:, 0:1]
        xf = xf_ref[b]
        sig = jnp.where(att >= tcol, 1.0 / (1.0 + jnp.exp(-att)), 0.0)
        sigd = 1.0 / (1.0 + jnp.exp(-diag_ref[b]))
        sig = jnp.where(eye, jnp.broadcast_to(sigd, (C, C)), sig)
        attx = jnp.dot(sig, xf, preferred_element_type=jnp.float32)
        outx = attx + xf
        y = jnp.dot(cw_ref[...], outx, preferred_element_type=jnp.float32)
        y_scr[b] = y
        tot = tot + jnp.sum(y, axis=1, keepdims=True)
        tot2 = tot2 + jnp.sum(y * y, axis=1, keepdims=True)
    n_inv = 1.0 / (B * HW)
    mean = tot * n_inv
    var = tot2 * n_inv - mean * mean
    scale = g_ref[...] * lax.rsqrt(var + EPS)
    shift = be_ref[...] - mean * scale
    for b in range(B):
        o_ref[b] = jnp.maximum(y_scr[b] * scale + shift, 0.0)


def _tc2():
    return pl.pallas_call(
        _tc2_body,
        in_specs=[
            pl.BlockSpec((B, C, C), lambda: (0, 0, 0)),
            pl.BlockSpec((B, C, 128), lambda: (0, 0, 0)),
            pl.BlockSpec((B, 1, C), lambda: (0, 0, 0)),
            pl.BlockSpec((B, C, HW), lambda: (0, 0, 0)),
            pl.BlockSpec((D, C), lambda: (0, 0)),
            pl.BlockSpec((D, 1), lambda: (0, 0)),
            pl.BlockSpec((D, 1), lambda: (0, 0)),
        ],
        out_specs=pl.BlockSpec((B, D, HW), lambda: (0, 0, 0)),
        out_shape=jax.ShapeDtypeStruct((B, D, HW), jnp.float32),
        scratch_shapes=[pltpu.VMEM((B, D, HW), jnp.float32)],
    )


@jax.jit
def kernel(x1, x2, Wq, bq, Wk, bk, conv_w, gamma, beta):
    krt = jnp.asarray(_KRT)
    x1f = x1.reshape(B, 384, HW)
    x2f = x2.reshape(B, 384, 144)
    xf, att_mod, diag = _tc1()(
        x1f, x2f, krt, Wq.T, Wk.T, bq.reshape(1, D), bk.reshape(1, D))
    t_flat = _sc_topk(att_mod.reshape(ROWS, C))
    t4 = t_flat.reshape(B, C, 128)
    out = _tc2()(att_mod, t4, diag, xf, conv_w,
                 gamma.reshape(D, 1), beta.reshape(D, 1))
    return out.reshape(B, D, 24, 24)


# speedup vs baseline: 1.6109x; 1.6109x over previous
import functools

import numpy as np
import jax
import jax.numpy as jnp
from jax import lax
from jax.experimental import pallas as pl
from jax.experimental.pallas import tpu as pltpu
from jax.experimental.pallas import tpu_sc as plsc

B = 4
C = 768
HW = 576
D = 256
K = 32
NEG = -1e9
SCALE_INV = 1.0 / 16.0
EPS = 1e-5
ROWS = B * C

_HIGH = jax.lax.Precision.HIGHEST


def _resize_matrix(out_n: int, in_n: int) -> np.ndarray:
    R = np.zeros((out_n, in_n), np.float32)
    for i in range(out_n):
        s = (i + 0.5) * in_n / out_n - 0.5
        i0 = int(np.floor(s))
        f = s - i0
        R[i, min(max(i0, 0), in_n - 1)] += 1.0 - f
        R[i, min(max(i0 + 1, 0), in_n - 1)] += f
    return R


_R24 = _resize_matrix(24, 12)
_KRT = np.kron(_R24, _R24).T.astype(np.float32)


def _tc1_body(x1_ref, x2_ref, krt_ref, wq_ref, wk_ref, bq_ref, bk_ref,
              xf_ref, att_ref, diag_ref):
    xa = x1_ref[0]
    xb = jnp.dot(x2_ref[0], krt_ref[...],
                 preferred_element_type=jnp.float32, precision=_HIGH)
    xf = jnp.concatenate([xa, xb], axis=0)
    xf_ref[0] = xf
    q = jnp.dot(xf, wq_ref[...],
                preferred_element_type=jnp.float32) + bq_ref[...]
    k = jnp.dot(xf, wk_ref[...],
                preferred_element_type=jnp.float32) + bk_ref[...]
    att = lax.dot_general(q, k, (((1,), (1,)), ((), ())),
                          preferred_element_type=jnp.float32) * SCALE_INV
    ri = lax.broadcasted_iota(jnp.int32, (C, C), 0)
    ci = lax.broadcasted_iota(jnp.int32, (C, C), 1)
    eye = ri == ci
    diag_ref[0] = jnp.sum(jnp.where(eye, att, 0.0), axis=0, keepdims=True)
    att_ref[0] = jnp.where(eye, NEG, att)


def _tc1():
    return pl.pallas_call(
        _tc1_body,
        grid=(B,),
        in_specs=[
            pl.BlockSpec((1, 384, HW), lambda b: (b, 0, 0)),
            pl.BlockSpec((1, 384, 144), lambda b: (b, 0, 0)),
            pl.BlockSpec((144, HW), lambda b: (0, 0)),
            pl.BlockSpec((HW, D), lambda b: (0, 0)),
            pl.BlockSpec((HW, D), lambda b: (0, 0)),
            pl.BlockSpec((1, D), lambda b: (0, 0)),
            pl.BlockSpec((1, D), lambda b: (0, 0)),
        ],
        out_specs=[
            pl.BlockSpec((1, C, HW), lambda b: (b, 0, 0)),
            pl.BlockSpec((1, C, C), lambda b: (b, 0, 0)),
            pl.BlockSpec((1, 1, C), lambda b: (b, 0, 0)),
        ],
        out_shape=[
            jax.ShapeDtypeStruct((B, C, HW), jnp.float32),
            jax.ShapeDtypeStruct((B, C, C), jnp.float32),
            jax.ShapeDtypeStruct((B, 1, C), jnp.float32),
        ],
    )


_NC = 2
_NS = 16
_NW = _NC * _NS
_RPW = ROWS // _NW


def _sort16(v):
    return lax.sort(v, dimension=0)


def _rev(v):
    return lax.rev(v, (0,))


def _merge_pair16(a, b):
    rb = _rev(b)
    return _sort16(jnp.minimum(a, rb)), _sort16(jnp.maximum(a, rb))


def _merge32(A, Bn):
    H0 = jnp.maximum(A[0], _rev(Bn[1]))
    H1 = jnp.maximum(A[1], _rev(Bn[0]))
    return _sort16(jnp.minimum(H0, H1)), _sort16(jnp.maximum(H0, H1))


def _row_threshold(att_v, r):
    chunks = [att_v[r, pl.ds(16 * j, 16)] for j in range(C // 16)]
    s = [_sort16(c) for c in chunks]
    nodes = [_merge_pair16(s[2 * i], s[2 * i + 1]) for i in range(len(s) // 2)]
    while len(nodes) > 2:
        new = [_merge32(nodes[2 * i], nodes[2 * i + 1])
               for i in range(len(nodes) // 2)]
        if len(nodes) % 2:
            new.append(nodes[-1])
        nodes = new
    A, Bn = nodes
    H0 = jnp.maximum(A[0], _rev(Bn[1]))
    H1 = jnp.maximum(A[1], _rev(Bn[0]))
    return jnp.min(jnp.minimum(H0, H1))


def _sc_topk_kernel(att_hbm, t_hbm, att_v, t_v):
    wid = lax.axis_index("s") * _NC + lax.axis_index("c")
    base = wid * _RPW
    pltpu.sync_copy(att_hbm.at[pl.ds(base, _RPW), :], att_v)

    def row_body(i, carry):
        for u in range(2):
            r = i * 2 + u
            t = _row_threshold(att_v, r)
            t_v[pl.ds(r * 128, 16)] = jnp.full((16,), t, jnp.float32)
        return carry

    lax.fori_loop(0, _RPW // 2, row_body, 0)
    pltpu.sync_copy(t_v, t_hbm.at[pl.ds(base * 128, _RPW * 128)])


def _sc_topk(att_flat):
    fn = functools.partial(
        pl.kernel,
        mesh=plsc.VectorSubcoreMesh(core_axis_name="c", subcore_axis_name="s"),
        compiler_params=pltpu.CompilerParams(needs_layout_passes=False,
                                             use_tc_tiling_on_sc=True),
        out_type=jax.ShapeDtypeStruct((ROWS * 128,), jnp.float32),
        scratch_types=[
            pltpu.VMEM((_RPW, C), jnp.float32),
            pltpu.VMEM((_RPW * 128,), jnp.float32),
        ],
    )(_sc_topk_kernel)
    return fn(att_flat)


def _tc2_body(att_ref, t_ref, diag_ref, xf_ref, cw_ref, g_ref, be_ref,
              o_ref, y_scr):
    ri = lax.broadcasted_iota(jnp.int32, (C, C), 0)
    ci = lax.broadcasted_iota(jnp.int32, (C, C), 1)
    eye = ri == ci
    tot = jnp.zeros((D, 1), jnp.float32)
    tot2 = jnp.zeros((D, 1), jnp.float32)
    for b in range(B):
        att = att_ref[b]
        tcol = t_ref[b][:, 0:1]
        xf = xf_ref[b]
        sig = jnp.where(att >= tcol, 1.0 / (1.0 + jnp.exp(-att)), 0.0)
        sigd = 1.0 / (1.0 + jnp.exp(-diag_ref[b]))
        sig = jnp.where(eye, jnp.broadcast_to(sigd, (C, C)), sig)
        attx = jnp.dot(sig, xf, preferred_element_type=jnp.float32)
        outx = attx + xf
        y = jnp.dot(cw_ref[...], outx, preferred_element_type=jnp.float32)
        y_scr[b] = y
        tot = tot + jnp.sum(y, axis=1, keepdims=True)
        tot2 = tot2 + jnp.sum(y * y, axis=1, keepdims=True)
    n_inv = 1.0 / (B * HW)
    mean = tot * n_inv
    var = tot2 * n_inv - mean * mean
    scale = g_ref[...] * lax.rsqrt(var + EPS)
    shift = be_ref[...] - mean * scale
    for b in range(B):
        o_ref[b] = jnp.maximum(y_scr[b] * scale + shift, 0.0)


def _tc2():
    return pl.pallas_call(
        _tc2_body,
        in_specs=[
            pl.BlockSpec((B, C, C), lambda: (0, 0, 0)),
            pl.BlockSpec((B, C, 128), lambda: (0, 0, 0)),
            pl.BlockSpec((B, 1, C), lambda: (0, 0, 0)),
            pl.BlockSpec((B, C, HW), lambda: (0, 0, 0)),
            pl.BlockSpec((D, C), lambda: (0, 0)),
            pl.BlockSpec((D, 1), lambda: (0, 0)),
            pl.BlockSpec((D, 1), lambda: (0, 0)),
        ],
        out_specs=pl.BlockSpec((B, D, HW), lambda: (0, 0, 0)),
        out_shape=jax.ShapeDtypeStruct((B, D, HW), jnp.float32),
        scratch_shapes=[pltpu.VMEM((B, D, HW), jnp.float32)],
    )


@jax.jit
def kernel(x1, x2, Wq, bq, Wk, bk, conv_w, gamma, beta):
    krt = jnp.asarray(_KRT)
    x1f = x1.reshape(B, 384, HW)
    x2f = x2.reshape(B, 384, 144)
    xf, att_mod, diag = _tc1()(
        x1f, x2f, krt, Wq.T, Wk.T, bq.reshape(1, D), bk.reshape(1, D))
    t_flat = _sc_topk(att_mod.reshape(ROWS, C))
    t4 = t_flat.reshape(B, C, 128)
    out = _tc2()(att_mod, t4, diag, xf, conv_w,
                 gamma.reshape(D, 1), beta.reshape(D, 1))
    return out.reshape(B, D, 24, 24)


# speedup vs baseline: 1.6423x; 1.0195x over previous
import functools

import numpy as np
import jax
import jax.numpy as jnp
from jax import lax
from jax.experimental import pallas as pl
from jax.experimental.pallas import tpu as pltpu
from jax.experimental.pallas import tpu_sc as plsc

B = 4
C = 768
HW = 576
D = 256
K = 32
NEG = -1e9
SCALE_INV = 1.0 / 16.0
EPS = 1e-5
ROWS = B * C

_HIGH = jax.lax.Precision.HIGHEST


def _resize_matrix(out_n: int, in_n: int) -> np.ndarray:
    R = np.zeros((out_n, in_n), np.float32)
    for i in range(out_n):
        s = (i + 0.5) * in_n / out_n - 0.5
        i0 = int(np.floor(s))
        f = s - i0
        R[i, min(max(i0, 0), in_n - 1)] += 1.0 - f
        R[i, min(max(i0 + 1, 0), in_n - 1)] += f
    return R


_R24 = _resize_matrix(24, 12)
_KRT = np.kron(_R24, _R24).T.astype(np.float32)


def _tc1_body(x1_ref, x2_ref, krt_ref, wq_ref, wk_ref, bq_ref, bk_ref,
              xf_ref, att_ref, diag_ref):
    xa = x1_ref[0]
    xb = jnp.dot(x2_ref[0], krt_ref[...],
                 preferred_element_type=jnp.float32, precision=_HIGH)
    xf = jnp.concatenate([xa, xb], axis=0)
    xf_ref[0] = xf
    q = jnp.dot(xf, wq_ref[...],
                preferred_element_type=jnp.float32) + bq_ref[...]
    k = jnp.dot(xf, wk_ref[...],
                preferred_element_type=jnp.float32) + bk_ref[...]
    att = lax.dot_general(q, k, (((1,), (1,)), ((), ())),
                          preferred_element_type=jnp.float32) * SCALE_INV
    ri = lax.broadcasted_iota(jnp.int32, (C, C), 0)
    ci = lax.broadcasted_iota(jnp.int32, (C, C), 1)
    eye = ri == ci
    diag_ref[0] = jnp.sum(jnp.where(eye, att, 0.0), axis=0, keepdims=True)
    att_ref[0] = jnp.where(eye, NEG, att)


def _tc1(nb):
    return pl.pallas_call(
        _tc1_body,
        grid=(nb,),
        in_specs=[
            pl.BlockSpec((1, 384, HW), lambda b: (b, 0, 0)),
            pl.BlockSpec((1, 384, 144), lambda b: (b, 0, 0)),
            pl.BlockSpec((144, HW), lambda b: (0, 0)),
            pl.BlockSpec((HW, D), lambda b: (0, 0)),
            pl.BlockSpec((HW, D), lambda b: (0, 0)),
            pl.BlockSpec((1, D), lambda b: (0, 0)),
            pl.BlockSpec((1, D), lambda b: (0, 0)),
        ],
        out_specs=[
            pl.BlockSpec((1, C, HW), lambda b: (b, 0, 0)),
            pl.BlockSpec((1, C, C), lambda b: (b, 0, 0)),
            pl.BlockSpec((1, 1, C), lambda b: (b, 0, 0)),
        ],
        out_shape=[
            jax.ShapeDtypeStruct((nb, C, HW), jnp.float32),
            jax.ShapeDtypeStruct((nb, C, C), jnp.float32),
            jax.ShapeDtypeStruct((nb, 1, C), jnp.float32),
        ],
    )


_NC = 2
_NS = 16
_NW = _NC * _NS
_RPW = ROWS // _NW


def _sort16(v):
    return lax.sort(v, dimension=0)


def _rev(v):
    return lax.rev(v, (0,))


def _merge_pair16(a, b):
    rb = _rev(b)
    return _sort16(jnp.minimum(a, rb)), _sort16(jnp.maximum(a, rb))


def _merge32(A, Bn):
    H0 = jnp.maximum(A[0], _rev(Bn[1]))
    H1 = jnp.maximum(A[1], _rev(Bn[0]))
    return _sort16(jnp.minimum(H0, H1)), _sort16(jnp.maximum(H0, H1))


def _row_threshold(att_v, r):
    chunks = [att_v[r, pl.ds(16 * j, 16)] for j in range(C // 16)]
    s = [_sort16(c) for c in chunks]
    nodes = [_merge_pair16(s[2 * i], s[2 * i + 1]) for i in range(len(s) // 2)]
    while len(nodes) > 2:
        new = [_merge32(nodes[2 * i], nodes[2 * i + 1])
               for i in range(len(nodes) // 2)]
        if len(nodes) % 2:
            new.append(nodes[-1])
        nodes = new
    A, Bn = nodes
    H0 = jnp.maximum(A[0], _rev(Bn[1]))
    H1 = jnp.maximum(A[1], _rev(Bn[0]))
    return jnp.min(jnp.minimum(H0, H1))


def _sc_topk(att2d):
    rows = att2d.shape[0]
    rpw = rows // _NW

    def body(att_hbm, t_hbm, att_v, t_v):
        wid = lax.axis_index("s") * _NC + lax.axis_index("c")
        base = wid * rpw
        pltpu.sync_copy(att_hbm.at[pl.ds(base, rpw), :], att_v)

        def row_body(i, carry):
            for u in range(2):
                r = i * 2 + u
                t = _row_threshold(att_v, r)
                t_v[pl.ds(r * 128, 16)] = jnp.full((16,), t, jnp.float32)
            return carry

        lax.fori_loop(0, rpw // 2, row_body, 0)
        pltpu.sync_copy(t_v, t_hbm.at[pl.ds(base * 128, rpw * 128)])

    fn = functools.partial(
        pl.kernel,
        mesh=plsc.VectorSubcoreMesh(core_axis_name="c", subcore_axis_name="s"),
        compiler_params=pltpu.CompilerParams(needs_layout_passes=False,
                                             use_tc_tiling_on_sc=True),
        out_type=jax.ShapeDtypeStruct((rows * 128,), jnp.float32),
        scratch_types=[
            pltpu.VMEM((rpw, C), jnp.float32),
            pltpu.VMEM((rpw * 128,), jnp.float32),
        ],
    )(body)
    return fn(att2d)


def _tc2_body(att_a, att_b, t_a, t_b, diag_a, diag_b, xf_a, xf_b,
              cw_ref, g_ref, be_ref, o_ref, y_scr):
    ri = lax.broadcasted_iota(jnp.int32, (C, C), 0)
    ci = lax.broadcasted_iota(jnp.int32, (C, C), 1)
    eye = ri == ci
    tot = jnp.zeros((D, 1), jnp.float32)
    tot2 = jnp.zeros((D, 1), jnp.float32)
    for b in range(B):
        att_ref, t_ref, diag_ref, xf_ref = (
            (att_a, t_a, diag_a, xf_a) if b < 2
            else (att_b, t_b, diag_b, xf_b))
        h = b % 2
        att = att_ref[h]
        tcol = t_ref[h][:, 0:1]
        xf = xf_ref[h]
        sig = jnp.where(att >= tcol, 1.0 / (1.0 + jnp.exp(-att)), 0.0)
        sigd = 1.0 / (1.0 + jnp.exp(-diag_ref[h]))
        sig = jnp.where(eye, jnp.broadcast_to(sigd, (C, C)), sig)
        attx = jnp.dot(sig, xf, preferred_element_type=jnp.float32)
        outx = attx + xf
        y = jnp.dot(cw_ref[...], outx, preferred_element_type=jnp.float32)
        y_scr[b] = y
        tot = tot + jnp.sum(y, axis=1, keepdims=True)
        tot2 = tot2 + jnp.sum(y * y, axis=1, keepdims=True)
    n_inv = 1.0 / (B * HW)
    mean = tot * n_inv
    var = tot2 * n_inv - mean * mean
    scale = g_ref[...] * lax.rsqrt(var + EPS)
    shift = be_ref[...] - mean * scale
    for b in range(B):
        o_ref[b] = jnp.maximum(y_scr[b] * scale + shift, 0.0)


def _tc2():
    return pl.pallas_call(
        _tc2_body,
        in_specs=[
            pl.BlockSpec((2, C, C), lambda: (0, 0, 0)),
            pl.BlockSpec((2, C, C), lambda: (0, 0, 0)),
            pl.BlockSpec((2, C, 128), lambda: (0, 0, 0)),
            pl.BlockSpec((2, C, 128), lambda: (0, 0, 0)),
            pl.BlockSpec((2, 1, C), lambda: (0, 0, 0)),
            pl.BlockSpec((2, 1, C), lambda: (0, 0, 0)),
            pl.BlockSpec((2, C, HW), lambda: (0, 0, 0)),
            pl.BlockSpec((2, C, HW), lambda: (0, 0, 0)),
            pl.BlockSpec((D, C), lambda: (0, 0)),
            pl.BlockSpec((D, 1), lambda: (0, 0)),
            pl.BlockSpec((D, 1), lambda: (0, 0)),
        ],
        out_specs=pl.BlockSpec((B, D, HW), lambda: (0, 0, 0)),
        out_shape=jax.ShapeDtypeStruct((B, D, HW), jnp.float32),
        scratch_shapes=[pltpu.VMEM((B, D, HW), jnp.float32)],
    )


@jax.jit
def kernel(x1, x2, Wq, bq, Wk, bk, conv_w, gamma, beta):
    krt = jnp.asarray(_KRT)
    x1f = x1.reshape(B, 384, HW)
    x2f = x2.reshape(B, 384, 144)
    wqt, wkt = Wq.T, Wk.T
    bq2, bk2 = bq.reshape(1, D), bk.reshape(1, D)
    xf_a, att_a, diag_a = _tc1(2)(x1f[:2], x2f[:2], krt, wqt, wkt, bq2, bk2)
    t_a = _sc_topk(att_a.reshape(2 * C, C))
    xf_b, att_b, diag_b = _tc1(2)(x1f[2:], x2f[2:], krt, wqt, wkt, bq2, bk2)
    t_b = _sc_topk(att_b.reshape(2 * C, C))
    out = _tc2()(att_a, att_b, t_a.reshape(2, C, 128), t_b.reshape(2, C, 128),
                 diag_a, diag_b, xf_a, xf_b, conv_w,
                 gamma.reshape(D, 1), beta.reshape(D, 1))
    return out.reshape(B, D, 24, 24)
